# baseline (speedup 1.0000x reference)
import functools

import jax
import jax.numpy as jnp
from jax import lax
from jax.experimental import pallas as pl
from jax.experimental.pallas import tpu as pltpu
from jax.experimental.pallas import tpu_sc as plsc

_NUM_CORES = 2
_NUM_SUBCORES = 16
_NUM_WORKERS = _NUM_CORES * _NUM_SUBCORES

_CHUNK = 8
_SLOTS = 7


def _embed(ids, table):
    BATCH, SEQ = ids.shape
    V, D = table.shape
    B = BATCH * SEQ
    b_per_w = B // _NUM_WORKERS
    n_chunks = b_per_w // _CHUNK

    mesh = plsc.VectorSubcoreMesh(core_axis_name="c", subcore_axis_name="s")

    @functools.partial(
        pl.kernel,
        mesh=mesh,
        out_type=jax.ShapeDtypeStruct((BATCH, SEQ, D), jnp.float32),
        scratch_types=[
            pltpu.VMEM((b_per_w,), jnp.int32),
            pltpu.VMEM((_SLOTS * _CHUNK, D), jnp.float32),
            pltpu.SemaphoreType.DMA,
            pltpu.SemaphoreType.DMA,
        ],
    )
    def body(ids_hbm, table_hbm, out_hbm, idx_v, bufs, gsem, wsem):
        wid = lax.axis_index("s") * _NUM_CORES + lax.axis_index("c")
        base = wid * b_per_w
        b = base // SEQ
        col = base - b * SEQ
        pltpu.sync_copy(ids_hbm.at[b, pl.ds(col, b_per_w)], idx_v)

        def idx_at(c):
            return idx_v.at[pl.ds(c * _CHUNK, _CHUNK)]

        def out_at(c):
            return out_hbm.at[b, pl.ds(col + c * _CHUNK, _CHUNK)]

        def buf_at(c):
            return bufs.at[pl.ds(lax.rem(c, _SLOTS) * _CHUNK, _CHUNK)]

        def gather_start(c):
            pltpu.async_copy(table_hbm.at[idx_at(c)], buf_at(c), gsem)

        def gather_wait(c):
            pltpu.make_async_copy(table_hbm.at[idx_at(c)], buf_at(c), gsem).wait()

        def write_start(c):
            pltpu.async_copy(buf_at(c), out_at(c), wsem)

        def write_wait(c):
            pltpu.make_async_copy(buf_at(c), out_at(c), wsem).wait()

        gather_start(0)
        gather_start(1)
        gather_start(2)

        def step(c, carry):
            @pl.when(c >= 4)
            def _():
                write_wait(c - 4)

            @pl.when(c + 3 < n_chunks)
            def _():
                gather_start(c + 3)

            gather_wait(c)
            write_start(c)
            return carry

        lax.fori_loop(0, n_chunks, step, 0, unroll=False)
        write_wait(n_chunks - 4)
        write_wait(n_chunks - 3)
        write_wait(n_chunks - 2)
        write_wait(n_chunks - 1)

    return body(ids, table)


def kernel(input_ids, table):
    return _embed(input_ids.astype(jnp.int32), table)


# speedup vs baseline: 1.0035x; 1.0035x over previous
import functools

import jax
import jax.numpy as jnp
from jax import lax
from jax.experimental import pallas as pl
from jax.experimental.pallas import tpu as pltpu
from jax.experimental.pallas import tpu_sc as plsc

_NUM_CORES = 2
_NUM_SUBCORES = 16
_NUM_WORKERS = _NUM_CORES * _NUM_SUBCORES

_CHUNK = 8
_SLOTS = 6


def _embed(ids, table):
    BATCH, SEQ = ids.shape
    V, D = table.shape
    B = BATCH * SEQ
    b_per_w = B // _NUM_WORKERS
    n_chunks = b_per_w // _CHUNK

    mesh = plsc.VectorSubcoreMesh(core_axis_name="c", subcore_axis_name="s")

    @functools.partial(
        pl.kernel,
        mesh=mesh,
        out_type=jax.ShapeDtypeStruct((BATCH, SEQ, D), jnp.float32),
        scratch_types=[
            pltpu.VMEM((b_per_w,), jnp.int32),
            pltpu.VMEM((_SLOTS * _CHUNK, D), jnp.float32),
            pltpu.SemaphoreType.DMA,
            pltpu.SemaphoreType.DMA,
        ],
    )
    def body(ids_hbm, table_hbm, out_hbm, idx_v, bufs, gsem, wsem):
        wid = lax.axis_index("s") * _NUM_CORES + lax.axis_index("c")
        base = wid * b_per_w
        b = base // SEQ
        col = base - b * SEQ
        pltpu.sync_copy(ids_hbm.at[b, pl.ds(col, b_per_w)], idx_v)

        def idx_at(c):
            return idx_v.at[pl.ds(c * _CHUNK, _CHUNK)]

        def buf_at(c):
            return bufs.at[pl.ds(lax.rem(c, _SLOTS) * _CHUNK, _CHUNK)]

        def pair_buf_at(p):
            return bufs.at[pl.ds(lax.rem(2 * p, _SLOTS) * _CHUNK, 2 * _CHUNK)]

        def pair_out_at(p):
            return out_hbm.at[b, pl.ds(col + p * 2 * _CHUNK, 2 * _CHUNK)]

        def gather_start(c):
            pltpu.async_copy(table_hbm.at[idx_at(c)], buf_at(c), gsem)

        def gather_wait(c):
            pltpu.make_async_copy(table_hbm.at[idx_at(c)], buf_at(c), gsem).wait()

        def write_start(p):
            pltpu.async_copy(pair_buf_at(p), pair_out_at(p), wsem)

        def write_wait(p):
            pltpu.make_async_copy(pair_buf_at(p), pair_out_at(p), wsem).wait()

        n_pairs = n_chunks // 2
        gather_start(0)
        gather_start(1)

        def step(p, carry):
            @pl.when(p >= 2)
            def _():
                write_wait(p - 2)

            c = 2 * p

            @pl.when(c + 2 < n_chunks)
            def _():
                gather_start(c + 2)

            @pl.when(c + 3 < n_chunks)
            def _():
                gather_start(c + 3)

            gather_wait(c)
            gather_wait(c + 1)
            write_start(p)
            return carry

        lax.fori_loop(0, n_pairs, step, 0, unroll=False)
        write_wait(n_pairs - 2)
        write_wait(n_pairs - 1)

    return body(ids, table)


def kernel(input_ids, table):
    return _embed(input_ids.astype(jnp.int32), table)
